# Initial kernel scaffold; baseline (speedup 1.0000x reference)
import jax
import jax.numpy as jnp
from jax.experimental import pallas as pl


def kernel(xyz, features, opacity, image_height, image_width, bg_color):
    raise NotImplementedError("write your pallas kernel here")


# speedup vs baseline: 83.5408x; 83.5408x over previous
import jax
import jax.numpy as jnp
from jax import lax
from jax.experimental import pallas as pl
from jax.experimental.pallas import tpu as pltpu

_N = 16384
_HW = 384
_ROWS = (_HW * _HW) // 128


def _prep_kernel(wh_ref, xs_ref, ys_ref, r_ref, g_ref, b_ref,
                 row_ref, lane_ref, cr_ref, cg_ref, cb_ref):
    x = xs_ref[...]
    y = ys_ref[...]
    wf = wh_ref[0]
    hf = wh_ref[1]
    wi = wf.astype(jnp.int32)
    hi = hf.astype(jnp.int32)

    xmin = jnp.min(x)
    xmax = jnp.max(x)
    ymin = jnp.min(y)
    ymax = jnp.max(y)

    xn = (x - xmin) / (xmax - xmin + 1e-08)
    yn = (y - ymin) / (ymax - ymin + 1e-08)
    xi = (xn * wf).astype(jnp.int32)
    yi = (yn * hf).astype(jnp.int32)
    valid = (xi >= 0) & (xi < wi) & (yi >= 0) & (yi < hi)
    xc = jnp.clip(xi, 0, wi - 1)
    yc = jnp.clip(yi, 0, hi - 1)

    p = yc * _HW + xc
    row = p // 128
    lane = p - row * 128
    row_ref[...] = jnp.where(valid, row, _ROWS)
    lane_ref[...] = lane.astype(jnp.float32)
    cr_ref[...] = jnp.clip(r_ref[...] + 0.5, 0.0, 1.0)
    cg_ref[...] = jnp.clip(g_ref[...] + 0.5, 0.0, 1.0)
    cb_ref[...] = jnp.clip(b_ref[...] + 0.5, 0.0, 1.0)


def _blend_kernel(row_ref, bg_ref, pts_ref, ci_ref, di_ref, ai_ref, fb_ref):
    for c in range(3):
        fb_ref[c, :, :] = jnp.full((_ROWS + 1, 128), bg_ref[c], jnp.float32)
    fb_ref[3, :, :] = jnp.zeros((_ROWS + 1, 128), jnp.float32)
    fb_ref[4, :, :] = jnp.zeros((_ROWS + 1, 128), jnp.float32)

    iota = lax.broadcasted_iota(jnp.int32, (1, 128), 1)

    def body(i, carry):
        rw = row_ref[i]
        v = pts_ref[pl.ds(i, 1), :]
        ln = v[:, 0:1].astype(jnp.int32)
        r = v[:, 1:2]
        g = v[:, 2:3]
        b = v[:, 3:4]
        a = v[:, 4:5]
        d = v[:, 5:6]
        m = iota == ln
        one_m_a = 1.0 - a

        cur = fb_ref[0, pl.ds(rw, 1), :]
        fb_ref[0, pl.ds(rw, 1), :] = jnp.where(m, a * r + one_m_a * cur, cur)
        cur = fb_ref[1, pl.ds(rw, 1), :]
        fb_ref[1, pl.ds(rw, 1), :] = jnp.where(m, a * g + one_m_a * cur, cur)
        cur = fb_ref[2, pl.ds(rw, 1), :]
        fb_ref[2, pl.ds(rw, 1), :] = jnp.where(m, a * b + one_m_a * cur, cur)
        cur = fb_ref[3, pl.ds(rw, 1), :]
        fb_ref[3, pl.ds(rw, 1), :] = jnp.where(m, a + one_m_a * cur, cur)
        cur = fb_ref[4, pl.ds(rw, 1), :]
        fb_ref[4, pl.ds(rw, 1), :] = jnp.where(m, d, cur)
        return carry

    lax.fori_loop(0, _N, body, 0)

    ci_ref[...] = fb_ref[0:3, 0:_ROWS, :]
    di_ref[...] = fb_ref[4:5, 0:_ROWS, :]
    ai_ref[...] = fb_ref[3:4, 0:_ROWS, :]


def kernel(xyz, features, opacity, image_height, image_width, bg_color):
    wh = jnp.stack([image_width, image_height]).astype(jnp.float32)
    xs = xyz[:, 0].reshape(128, 128)
    ys = xyz[:, 1].reshape(128, 128)
    r0 = features[:, 0, 0].reshape(128, 128)
    g0 = features[:, 0, 1].reshape(128, 128)
    b0 = features[:, 0, 2].reshape(128, 128)

    row, lanef, cr, cg, cb = pl.pallas_call(
        _prep_kernel,
        in_specs=[pl.BlockSpec(memory_space=pltpu.SMEM)]
        + [pl.BlockSpec((128, 128), lambda: (0, 0))] * 5,
        out_shape=[
            jax.ShapeDtypeStruct((128, 128), jnp.int32),
            jax.ShapeDtypeStruct((128, 128), jnp.float32),
            jax.ShapeDtypeStruct((128, 128), jnp.float32),
            jax.ShapeDtypeStruct((128, 128), jnp.float32),
            jax.ShapeDtypeStruct((128, 128), jnp.float32),
        ],
    )(wh, xs, ys, r0, g0, b0)

    pts = jnp.stack(
        [lanef.reshape(_N), cr.reshape(_N), cg.reshape(_N), cb.reshape(_N),
         opacity[:, 0], xyz[:, 2]],
        axis=-1,
    )

    ci, di, ai = pl.pallas_call(
        _blend_kernel,
        in_specs=[
            pl.BlockSpec(memory_space=pltpu.SMEM),
            pl.BlockSpec(memory_space=pltpu.SMEM),
            pl.BlockSpec((_N, 6), lambda: (0, 0)),
        ],
        out_shape=[
            jax.ShapeDtypeStruct((3, _ROWS, 128), jnp.float32),
            jax.ShapeDtypeStruct((1, _ROWS, 128), jnp.float32),
            jax.ShapeDtypeStruct((1, _ROWS, 128), jnp.float32),
        ],
        scratch_shapes=[pltpu.VMEM((5, _ROWS + 1, 128), jnp.float32)],
    )(row.reshape(_N), bg_color, pts)

    color_img = ci.reshape(3, _HW, _HW)
    depth_img = di.reshape(1, _HW, _HW)
    alpha_img = ai.reshape(1, _HW, _HW)
    return color_img, depth_img, alpha_img


# speedup vs baseline: 491.7543x; 5.8864x over previous
import functools

import jax
import jax.numpy as jnp
from jax import lax
from jax.experimental import pallas as pl
from jax.experimental.pallas import tpu as pltpu
from jax.experimental.pallas import tpu_sc as plsc

_N = 16384
_HW = 384
_NPIX = _HW * _HW
_NW = 32
_PPW = _NPIX // _NW
_SENT = 1 << 20
_CHUNKS = _N // 16


def _prep_kernel(wh_ref, xs_ref, ys_ref, r_ref, g_ref, b_ref,
                 pix_ref, cr_ref, cg_ref, cb_ref):
    x = xs_ref[...]
    y = ys_ref[...]
    wf = wh_ref[0]
    hf = wh_ref[1]
    wi = wf.astype(jnp.int32)
    hi = hf.astype(jnp.int32)

    xmin = jnp.min(x)
    xmax = jnp.max(x)
    ymin = jnp.min(y)
    ymax = jnp.max(y)

    xn = (x - xmin) / (xmax - xmin + 1e-08)
    yn = (y - ymin) / (ymax - ymin + 1e-08)
    xi = (xn * wf).astype(jnp.int32)
    yi = (yn * hf).astype(jnp.int32)
    valid = (xi >= 0) & (xi < wi) & (yi >= 0) & (yi < hi)
    xc = jnp.clip(xi, 0, wi - 1)
    yc = jnp.clip(yi, 0, hi - 1)

    p = yc * _HW + xc
    pix_ref[...] = jnp.where(valid, p, _SENT)
    cr_ref[...] = jnp.clip(r_ref[...] + 0.5, 0.0, 1.0)
    cg_ref[...] = jnp.clip(g_ref[...] + 0.5, 0.0, 1.0)
    cb_ref[...] = jnp.clip(b_ref[...] + 0.5, 0.0, 1.0)


def _sc_blend(pix_hbm, cr_hbm, cg_hbm, cb_hbm, a_hbm, d_hbm, bg_hbm,
              outr, outg, outb, outa, outd,
              pixv, crv, cgv, cbv, av, dv, bgv,
              fbr, fbg, fbb, fba, fbd):
    wid = lax.axis_index("s") * 2 + lax.axis_index("c")
    lo = wid * _PPW

    pltpu.sync_copy(pix_hbm, pixv)
    pltpu.sync_copy(cr_hbm, crv)
    pltpu.sync_copy(cg_hbm, cgv)
    pltpu.sync_copy(cb_hbm, cbv)
    pltpu.sync_copy(a_hbm, av)
    pltpu.sync_copy(d_hbm, dv)
    pltpu.sync_copy(bg_hbm, bgv)

    bgvec = bgv[...]
    bg_r = bgvec[0]
    bg_g = bgvec[1]
    bg_b = bgvec[2]

    def init_body(j, c):
        o = j * 16
        fbr[pl.ds(o, 16)] = jnp.full((16,), bg_r, jnp.float32)
        fbg[pl.ds(o, 16)] = jnp.full((16,), bg_g, jnp.float32)
        fbb[pl.ds(o, 16)] = jnp.full((16,), bg_b, jnp.float32)
        fba[pl.ds(o, 16)] = jnp.zeros((16,), jnp.float32)
        fbd[pl.ds(o, 16)] = jnp.zeros((16,), jnp.float32)
        return c

    lax.fori_loop(0, _PPW // 16, init_body, 0)

    lanes = lax.iota(jnp.int32, 16)

    def chunk_body(k, c):
        base = k * 16
        pv = pixv[pl.ds(base, 16)]
        owned = (pv >= lo) & (pv < lo + _PPW)

        n_owned = plsc.all_reduce_population_count(owned)[0]

        @pl.when(n_owned > 0)
        def _():
            local = jnp.clip(pv - lo, 0, _PPW - 1)
            r = crv[pl.ds(base, 16)]
            g = cgv[pl.ds(base, 16)]
            b = cbv[pl.ds(base, 16)]
            a = av[pl.ds(base, 16)]
            d = dv[pl.ds(base, 16)]
            one_m_a = 1.0 - a

            def lane_body(j, cc):
                mj = owned & (lanes == j)
                n_j = plsc.all_reduce_population_count(mj)[0]

                @pl.when(n_j > 0)
                def _():
                    cur = plsc.load_gather(fbr, [local], mask=mj)
                    plsc.store_scatter(fbr, [local], a * r + one_m_a * cur,
                                       mask=mj)
                    cur = plsc.load_gather(fbg, [local], mask=mj)
                    plsc.store_scatter(fbg, [local], a * g + one_m_a * cur,
                                       mask=mj)
                    cur = plsc.load_gather(fbb, [local], mask=mj)
                    plsc.store_scatter(fbb, [local], a * b + one_m_a * cur,
                                       mask=mj)
                    cur = plsc.load_gather(fba, [local], mask=mj)
                    plsc.store_scatter(fba, [local], a + one_m_a * cur,
                                       mask=mj)
                    plsc.store_scatter(fbd, [local], d, mask=mj)

                return cc

            lax.fori_loop(0, 16, lane_body, 0)

        return c

    lax.fori_loop(0, _CHUNKS, chunk_body, 0)

    pltpu.sync_copy(fbr, outr.at[pl.ds(lo, _PPW)])
    pltpu.sync_copy(fbg, outg.at[pl.ds(lo, _PPW)])
    pltpu.sync_copy(fbb, outb.at[pl.ds(lo, _PPW)])
    pltpu.sync_copy(fba, outa.at[pl.ds(lo, _PPW)])
    pltpu.sync_copy(fbd, outd.at[pl.ds(lo, _PPW)])


def kernel(xyz, features, opacity, image_height, image_width, bg_color):
    wh = jnp.stack([image_width, image_height]).astype(jnp.float32)
    xs = xyz[:, 0].reshape(128, 128)
    ys = xyz[:, 1].reshape(128, 128)
    r0 = features[:, 0, 0].reshape(128, 128)
    g0 = features[:, 0, 1].reshape(128, 128)
    b0 = features[:, 0, 2].reshape(128, 128)

    pix, cr, cg, cb = pl.pallas_call(
        _prep_kernel,
        in_specs=[pl.BlockSpec(memory_space=pltpu.SMEM)]
        + [pl.BlockSpec((128, 128), lambda: (0, 0))] * 5,
        out_shape=[
            jax.ShapeDtypeStruct((128, 128), jnp.int32),
            jax.ShapeDtypeStruct((128, 128), jnp.float32),
            jax.ShapeDtypeStruct((128, 128), jnp.float32),
            jax.ShapeDtypeStruct((128, 128), jnp.float32),
        ],
    )(wh, xs, ys, r0, g0, b0)

    bg16 = jnp.concatenate([bg_color, jnp.zeros((13,), jnp.float32)])

    blend = functools.partial(
        pl.kernel,
        out_type=[jax.ShapeDtypeStruct((_NPIX,), jnp.float32)] * 5,
        mesh=plsc.VectorSubcoreMesh(core_axis_name="c", subcore_axis_name="s",
                                    num_cores=2, num_subcores=16),
        compiler_params=pltpu.CompilerParams(needs_layout_passes=False),
        scratch_types=[
            pltpu.VMEM((_N,), jnp.int32),
            pltpu.VMEM((_N,), jnp.float32),
            pltpu.VMEM((_N,), jnp.float32),
            pltpu.VMEM((_N,), jnp.float32),
            pltpu.VMEM((_N,), jnp.float32),
            pltpu.VMEM((_N,), jnp.float32),
            pltpu.VMEM((16,), jnp.float32),
            pltpu.VMEM((_PPW,), jnp.float32),
            pltpu.VMEM((_PPW,), jnp.float32),
            pltpu.VMEM((_PPW,), jnp.float32),
            pltpu.VMEM((_PPW,), jnp.float32),
            pltpu.VMEM((_PPW,), jnp.float32),
        ],
    )(_sc_blend)

    outr, outg, outb, outa, outd = blend(
        pix.reshape(_N), cr.reshape(_N), cg.reshape(_N), cb.reshape(_N),
        opacity[:, 0], xyz[:, 2], bg16)

    color_img = jnp.stack([outr, outg, outb]).reshape(3, _HW, _HW)
    depth_img = outd.reshape(1, _HW, _HW)
    alpha_img = outa.reshape(1, _HW, _HW)
    return color_img, depth_img, alpha_img


# speedup vs baseline: 729.0549x; 1.4826x over previous
import functools

import jax
import jax.numpy as jnp
from jax import lax
from jax.experimental import pallas as pl
from jax.experimental.pallas import tpu as pltpu
from jax.experimental.pallas import tpu_sc as plsc

_N = 16384
_HW = 384
_NPIX = _HW * _HW
_NW = 32
_PPW = _NPIX // _NW
_SENT = 1 << 20
_CHUNKS = _N // 16


def _prep_kernel(wh_ref, xs_ref, ys_ref, r_ref, g_ref, b_ref,
                 pix_ref, cr_ref, cg_ref, cb_ref):
    x = xs_ref[...]
    y = ys_ref[...]
    wf = wh_ref[0]
    hf = wh_ref[1]
    wi = wf.astype(jnp.int32)
    hi = hf.astype(jnp.int32)

    xmin = jnp.min(x)
    xmax = jnp.max(x)
    ymin = jnp.min(y)
    ymax = jnp.max(y)

    xn = (x - xmin) / (xmax - xmin + 1e-08)
    yn = (y - ymin) / (ymax - ymin + 1e-08)
    xi = (xn * wf).astype(jnp.int32)
    yi = (yn * hf).astype(jnp.int32)
    valid = (xi >= 0) & (xi < wi) & (yi >= 0) & (yi < hi)
    xc = jnp.clip(xi, 0, wi - 1)
    yc = jnp.clip(yi, 0, hi - 1)

    p = yc * _HW + xc
    pix_ref[...] = jnp.where(valid, p, _SENT)
    cr_ref[...] = jnp.clip(r_ref[...] + 0.5, 0.0, 1.0)
    cg_ref[...] = jnp.clip(g_ref[...] + 0.5, 0.0, 1.0)
    cb_ref[...] = jnp.clip(b_ref[...] + 0.5, 0.0, 1.0)


def _sc_blend(pix_hbm, cr_hbm, cg_hbm, cb_hbm, a_hbm, d_hbm, bg_hbm,
              outr, outg, outb, outa, outd,
              pixv, crv, cgv, cbv, av, dv, bgv,
              fbr, fbg, fbb, fba, fbd):
    wid = lax.axis_index("s") * 2 + lax.axis_index("c")
    lo = wid * _PPW

    pltpu.sync_copy(pix_hbm, pixv)
    pltpu.sync_copy(cr_hbm, crv)
    pltpu.sync_copy(cg_hbm, cgv)
    pltpu.sync_copy(cb_hbm, cbv)
    pltpu.sync_copy(a_hbm, av)
    pltpu.sync_copy(d_hbm, dv)
    pltpu.sync_copy(bg_hbm, bgv)

    bgvec = bgv[...]
    bg_r = bgvec[0]
    bg_g = bgvec[1]
    bg_b = bgvec[2]

    def init_body(j, c):
        o = j * 16
        fbr[pl.ds(o, 16)] = jnp.full((16,), bg_r, jnp.float32)
        fbg[pl.ds(o, 16)] = jnp.full((16,), bg_g, jnp.float32)
        fbb[pl.ds(o, 16)] = jnp.full((16,), bg_b, jnp.float32)
        fba[pl.ds(o, 16)] = jnp.zeros((16,), jnp.float32)
        fbd[pl.ds(o, 16)] = jnp.zeros((16,), jnp.float32)
        return c

    lax.fori_loop(0, _PPW // 16, init_body, 0)

    lanes = lax.iota(jnp.int32, 16)

    def chunk_body(k, c):
        base = k * 16
        pv = pixv[pl.ds(base, 16)]
        owned = (pv >= lo) & (pv < lo + _PPW)

        n_owned = plsc.all_reduce_population_count(owned)[0]

        @pl.when(n_owned > 0)
        def _():
            local = jnp.clip(pv - lo, 0, _PPW - 1)
            r = crv[pl.ds(base, 16)]
            g = cgv[pl.ds(base, 16)]
            b = cbv[pl.ds(base, 16)]
            a = av[pl.ds(base, 16)]
            d = dv[pl.ds(base, 16)]
            one_m_a = 1.0 - a

            def blend_masked(mj):
                cur = plsc.load_gather(fbr, [local], mask=mj)
                plsc.store_scatter(fbr, [local], a * r + one_m_a * cur,
                                   mask=mj)
                cur = plsc.load_gather(fbg, [local], mask=mj)
                plsc.store_scatter(fbg, [local], a * g + one_m_a * cur,
                                   mask=mj)
                cur = plsc.load_gather(fbb, [local], mask=mj)
                plsc.store_scatter(fbb, [local], a * b + one_m_a * cur,
                                   mask=mj)
                cur = plsc.load_gather(fba, [local], mask=mj)
                plsc.store_scatter(fba, [local], a + one_m_a * cur,
                                   mask=mj)
                plsc.store_scatter(fbd, [local], d, mask=mj)

            @pl.when(n_owned == 1)
            def _():
                blend_masked(owned)

            @pl.when(n_owned > 1)
            def _():
                def lane_body(j, cc):
                    mj = owned & (lanes == j)
                    n_j = plsc.all_reduce_population_count(mj)[0]

                    @pl.when(n_j > 0)
                    def _():
                        blend_masked(mj)

                    return cc

                lax.fori_loop(0, 16, lane_body, 0)

        return c

    lax.fori_loop(0, _CHUNKS, chunk_body, 0)

    pltpu.sync_copy(fbr, outr.at[pl.ds(lo, _PPW)])
    pltpu.sync_copy(fbg, outg.at[pl.ds(lo, _PPW)])
    pltpu.sync_copy(fbb, outb.at[pl.ds(lo, _PPW)])
    pltpu.sync_copy(fba, outa.at[pl.ds(lo, _PPW)])
    pltpu.sync_copy(fbd, outd.at[pl.ds(lo, _PPW)])


def kernel(xyz, features, opacity, image_height, image_width, bg_color):
    wh = jnp.stack([image_width, image_height]).astype(jnp.float32)
    xs = xyz[:, 0].reshape(128, 128)
    ys = xyz[:, 1].reshape(128, 128)
    r0 = features[:, 0, 0].reshape(128, 128)
    g0 = features[:, 0, 1].reshape(128, 128)
    b0 = features[:, 0, 2].reshape(128, 128)

    pix, cr, cg, cb = pl.pallas_call(
        _prep_kernel,
        in_specs=[pl.BlockSpec(memory_space=pltpu.SMEM)]
        + [pl.BlockSpec((128, 128), lambda: (0, 0))] * 5,
        out_shape=[
            jax.ShapeDtypeStruct((128, 128), jnp.int32),
            jax.ShapeDtypeStruct((128, 128), jnp.float32),
            jax.ShapeDtypeStruct((128, 128), jnp.float32),
            jax.ShapeDtypeStruct((128, 128), jnp.float32),
        ],
    )(wh, xs, ys, r0, g0, b0)

    bg16 = jnp.concatenate([bg_color, jnp.zeros((13,), jnp.float32)])

    blend = functools.partial(
        pl.kernel,
        out_type=[jax.ShapeDtypeStruct((_NPIX,), jnp.float32)] * 5,
        mesh=plsc.VectorSubcoreMesh(core_axis_name="c", subcore_axis_name="s",
                                    num_cores=2, num_subcores=16),
        compiler_params=pltpu.CompilerParams(needs_layout_passes=False),
        scratch_types=[
            pltpu.VMEM((_N,), jnp.int32),
            pltpu.VMEM((_N,), jnp.float32),
            pltpu.VMEM((_N,), jnp.float32),
            pltpu.VMEM((_N,), jnp.float32),
            pltpu.VMEM((_N,), jnp.float32),
            pltpu.VMEM((_N,), jnp.float32),
            pltpu.VMEM((16,), jnp.float32),
            pltpu.VMEM((_PPW,), jnp.float32),
            pltpu.VMEM((_PPW,), jnp.float32),
            pltpu.VMEM((_PPW,), jnp.float32),
            pltpu.VMEM((_PPW,), jnp.float32),
            pltpu.VMEM((_PPW,), jnp.float32),
        ],
    )(_sc_blend)

    outr, outg, outb, outa, outd = blend(
        pix.reshape(_N), cr.reshape(_N), cg.reshape(_N), cb.reshape(_N),
        opacity[:, 0], xyz[:, 2], bg16)

    color_img = jnp.stack([outr, outg, outb]).reshape(3, _HW, _HW)
    depth_img = outd.reshape(1, _HW, _HW)
    alpha_img = outa.reshape(1, _HW, _HW)
    return color_img, depth_img, alpha_img


# speedup vs baseline: 2195.5154x; 3.0115x over previous
import functools

import jax
import jax.numpy as jnp
from jax import lax
from jax.experimental import pallas as pl
from jax.experimental.pallas import tpu as pltpu
from jax.experimental.pallas import tpu_sc as plsc

_N = 16384
_HW = 384
_NPIX = _HW * _HW
_NW = 32
_PPW = _NPIX // _NW
_SENT = 1 << 20
_CHUNKS = _N // 16


def _prep_kernel(wh_ref, xs_ref, ys_ref, r_ref, g_ref, b_ref,
                 pix_ref, cr_ref, cg_ref, cb_ref):
    x = xs_ref[...]
    y = ys_ref[...]
    wf = wh_ref[0]
    hf = wh_ref[1]
    wi = wf.astype(jnp.int32)
    hi = hf.astype(jnp.int32)

    xmin = jnp.min(x)
    xmax = jnp.max(x)
    ymin = jnp.min(y)
    ymax = jnp.max(y)

    xn = (x - xmin) / (xmax - xmin + 1e-08)
    yn = (y - ymin) / (ymax - ymin + 1e-08)
    xi = (xn * wf).astype(jnp.int32)
    yi = (yn * hf).astype(jnp.int32)
    valid = (xi >= 0) & (xi < wi) & (yi >= 0) & (yi < hi)
    xc = jnp.clip(xi, 0, wi - 1)
    yc = jnp.clip(yi, 0, hi - 1)

    p = yc * _HW + xc
    pix_ref[...] = jnp.where(valid, p, _SENT)
    cr_ref[...] = jnp.clip(r_ref[...] + 0.5, 0.0, 1.0)
    cg_ref[...] = jnp.clip(g_ref[...] + 0.5, 0.0, 1.0)
    cb_ref[...] = jnp.clip(b_ref[...] + 0.5, 0.0, 1.0)


_SB = 4096
_SB_CHUNKS = _SB // 16
_CB = _SB + 16


def _sc_blend(pix_hbm, cr_hbm, cg_hbm, cb_hbm, a_hbm, d_hbm, bg_hbm,
              outr, outg, outb, outa, outd,
              pixv, crv, cgv, cbv, av, dv, bgv,
              cloc, cr2, cg2, cb2, ca2, cd2, tmp,
              fbr, fbg, fbb, fba, fbd):
    wid = lax.axis_index("s") * 2 + lax.axis_index("c")
    lo = wid * _PPW

    pltpu.sync_copy(bg_hbm, bgv)
    bgvec = bgv[...]
    bg_r = bgvec[0]
    bg_g = bgvec[1]
    bg_b = bgvec[2]

    def init_body(j, c):
        o = j * 16
        fbr[pl.ds(o, 16)] = jnp.full((16,), bg_r, jnp.float32)
        fbg[pl.ds(o, 16)] = jnp.full((16,), bg_g, jnp.float32)
        fbb[pl.ds(o, 16)] = jnp.full((16,), bg_b, jnp.float32)
        fba[pl.ds(o, 16)] = jnp.zeros((16,), jnp.float32)
        fbd[pl.ds(o, 16)] = jnp.zeros((16,), jnp.float32)
        return c

    lax.fori_loop(0, _PPW // 16, init_body, 0)

    lanes = lax.iota(jnp.int32, 16)

    for sb in range(_N // _SB):
        off = sb * _SB
        pltpu.sync_copy(pix_hbm.at[pl.ds(off, _SB)], pixv)
        pltpu.sync_copy(cr_hbm.at[pl.ds(off, _SB)], crv)
        pltpu.sync_copy(cg_hbm.at[pl.ds(off, _SB)], cgv)
        pltpu.sync_copy(cb_hbm.at[pl.ds(off, _SB)], cbv)
        pltpu.sync_copy(a_hbm.at[pl.ds(off, _SB)], av)
        pltpu.sync_copy(d_hbm.at[pl.ds(off, _SB)], dv)

        def scan_body(k, cnt):
            base = k * 16
            pv = pixv[pl.ds(base, 16)]
            owned = (pv >= lo) & (pv < lo + _PPW)
            local = jnp.clip(pv - lo, 0, _PPW - 1)
            plsc.store_compressed(cloc.at[pl.ds(cnt, 16)], local, mask=owned)
            plsc.store_compressed(cr2.at[pl.ds(cnt, 16)],
                                  crv[pl.ds(base, 16)], mask=owned)
            plsc.store_compressed(cg2.at[pl.ds(cnt, 16)],
                                  cgv[pl.ds(base, 16)], mask=owned)
            plsc.store_compressed(cb2.at[pl.ds(cnt, 16)],
                                  cbv[pl.ds(base, 16)], mask=owned)
            plsc.store_compressed(ca2.at[pl.ds(cnt, 16)],
                                  av[pl.ds(base, 16)], mask=owned)
            plsc.store_compressed(cd2.at[pl.ds(cnt, 16)],
                                  dv[pl.ds(base, 16)], mask=owned)
            return cnt + plsc.all_reduce_population_count(owned)[0]

        cnt = lax.fori_loop(0, _SB_CHUNKS, scan_body, 0)

        def blend_body(t, c):
            base = t * 16
            m = (base + lanes) < cnt
            local = jnp.clip(cloc[pl.ds(base, 16)], 0, _PPW - 1)
            r = cr2[pl.ds(base, 16)]
            g = cg2[pl.ds(base, 16)]
            b = cb2[pl.ds(base, 16)]
            a = ca2[pl.ds(base, 16)]
            d = cd2[pl.ds(base, 16)]
            one_m_a = 1.0 - a

            def blend_masked(mj):
                cur = plsc.load_gather(fbr, [local], mask=mj)
                plsc.store_scatter(fbr, [local], a * r + one_m_a * cur,
                                   mask=mj)
                cur = plsc.load_gather(fbg, [local], mask=mj)
                plsc.store_scatter(fbg, [local], a * g + one_m_a * cur,
                                   mask=mj)
                cur = plsc.load_gather(fbb, [local], mask=mj)
                plsc.store_scatter(fbb, [local], a * b + one_m_a * cur,
                                   mask=mj)
                cur = plsc.load_gather(fba, [local], mask=mj)
                plsc.store_scatter(fba, [local], a + one_m_a * cur,
                                   mask=mj)
                plsc.store_scatter(fbd, [local], d, mask=mj)

            plsc.store_scatter(tmp, [local], lanes, mask=m)
            back = plsc.load_gather(tmp, [local], mask=m)
            ndup = plsc.all_reduce_population_count((back != lanes) & m)[0]

            @pl.when(ndup == 0)
            def _():
                blend_masked(m)

            @pl.when(ndup > 0)
            def _():
                def lane_body(j, cc):
                    mj = m & (lanes == j)
                    n_j = plsc.all_reduce_population_count(mj)[0]

                    @pl.when(n_j > 0)
                    def _():
                        blend_masked(mj)

                    return cc

                lax.fori_loop(0, 16, lane_body, 0)

            return c

        lax.fori_loop(0, (cnt + 15) // 16, blend_body, 0)

    pltpu.sync_copy(fbr, outr.at[pl.ds(lo, _PPW)])
    pltpu.sync_copy(fbg, outg.at[pl.ds(lo, _PPW)])
    pltpu.sync_copy(fbb, outb.at[pl.ds(lo, _PPW)])
    pltpu.sync_copy(fba, outa.at[pl.ds(lo, _PPW)])
    pltpu.sync_copy(fbd, outd.at[pl.ds(lo, _PPW)])


def kernel(xyz, features, opacity, image_height, image_width, bg_color):
    wh = jnp.stack([image_width, image_height]).astype(jnp.float32)
    xs = xyz[:, 0].reshape(128, 128)
    ys = xyz[:, 1].reshape(128, 128)
    r0 = features[:, 0, 0].reshape(128, 128)
    g0 = features[:, 0, 1].reshape(128, 128)
    b0 = features[:, 0, 2].reshape(128, 128)

    pix, cr, cg, cb = pl.pallas_call(
        _prep_kernel,
        in_specs=[pl.BlockSpec(memory_space=pltpu.SMEM)]
        + [pl.BlockSpec((128, 128), lambda: (0, 0))] * 5,
        out_shape=[
            jax.ShapeDtypeStruct((128, 128), jnp.int32),
            jax.ShapeDtypeStruct((128, 128), jnp.float32),
            jax.ShapeDtypeStruct((128, 128), jnp.float32),
            jax.ShapeDtypeStruct((128, 128), jnp.float32),
        ],
    )(wh, xs, ys, r0, g0, b0)

    bg16 = jnp.concatenate([bg_color, jnp.zeros((13,), jnp.float32)])

    blend = functools.partial(
        pl.kernel,
        out_type=[jax.ShapeDtypeStruct((_NPIX,), jnp.float32)] * 5,
        mesh=plsc.VectorSubcoreMesh(core_axis_name="c", subcore_axis_name="s",
                                    num_cores=2, num_subcores=16),
        compiler_params=pltpu.CompilerParams(needs_layout_passes=False),
        scratch_types=[
            pltpu.VMEM((_SB,), jnp.int32),
            pltpu.VMEM((_SB,), jnp.float32),
            pltpu.VMEM((_SB,), jnp.float32),
            pltpu.VMEM((_SB,), jnp.float32),
            pltpu.VMEM((_SB,), jnp.float32),
            pltpu.VMEM((_SB,), jnp.float32),
            pltpu.VMEM((16,), jnp.float32),
            pltpu.VMEM((_CB,), jnp.int32),
            pltpu.VMEM((_CB,), jnp.float32),
            pltpu.VMEM((_CB,), jnp.float32),
            pltpu.VMEM((_CB,), jnp.float32),
            pltpu.VMEM((_CB,), jnp.float32),
            pltpu.VMEM((_CB,), jnp.float32),
            pltpu.VMEM((_PPW,), jnp.int32),
            pltpu.VMEM((_PPW,), jnp.float32),
            pltpu.VMEM((_PPW,), jnp.float32),
            pltpu.VMEM((_PPW,), jnp.float32),
            pltpu.VMEM((_PPW,), jnp.float32),
            pltpu.VMEM((_PPW,), jnp.float32),
        ],
    )(_sc_blend)

    outr, outg, outb, outa, outd = blend(
        pix.reshape(_N), cr.reshape(_N), cg.reshape(_N), cb.reshape(_N),
        opacity[:, 0], xyz[:, 2], bg16)

    color_img = jnp.stack([outr, outg, outb]).reshape(3, _HW, _HW)
    depth_img = outd.reshape(1, _HW, _HW)
    alpha_img = outa.reshape(1, _HW, _HW)
    return color_img, depth_img, alpha_img


# speedup vs baseline: 3545.7417x; 1.6150x over previous
import functools

import jax
import jax.numpy as jnp
from jax import lax
from jax.experimental import pallas as pl
from jax.experimental.pallas import tpu as pltpu
from jax.experimental.pallas import tpu_sc as plsc

_N = 16384
_HW = 384
_NPIX = _HW * _HW
_NW = 32
_PPW = _NPIX // _NW
_SENT = 1 << 20
_CHUNKS = _N // 16


def _prep_kernel(wh_ref, xs_ref, ys_ref, r_ref, g_ref, b_ref,
                 pix_ref, cr_ref, cg_ref, cb_ref):
    x = xs_ref[...]
    y = ys_ref[...]
    wf = wh_ref[0]
    hf = wh_ref[1]
    wi = wf.astype(jnp.int32)
    hi = hf.astype(jnp.int32)

    xmin = jnp.min(x)
    xmax = jnp.max(x)
    ymin = jnp.min(y)
    ymax = jnp.max(y)

    xn = (x - xmin) / (xmax - xmin + 1e-08)
    yn = (y - ymin) / (ymax - ymin + 1e-08)
    xi = (xn * wf).astype(jnp.int32)
    yi = (yn * hf).astype(jnp.int32)
    valid = (xi >= 0) & (xi < wi) & (yi >= 0) & (yi < hi)
    xc = jnp.clip(xi, 0, wi - 1)
    yc = jnp.clip(yi, 0, hi - 1)

    p = yc * _HW + xc
    pix_ref[...] = jnp.where(valid, p, _SENT)
    cr_ref[...] = jnp.clip(r_ref[...] + 0.5, 0.0, 1.0)
    cg_ref[...] = jnp.clip(g_ref[...] + 0.5, 0.0, 1.0)
    cb_ref[...] = jnp.clip(b_ref[...] + 0.5, 0.0, 1.0)


_SB = 4096
_SB_CHUNKS = _SB // 16
_CB = _SB + 16


def _sc_blend(pix_hbm, cr_hbm, cg_hbm, cb_hbm, a_hbm, d_hbm, bg_hbm,
              outc, outa, outd,
              pixv0, crv0, cgv0, cbv0, av0, dv0,
              pixv1, crv1, cgv1, cbv1, av1, dv1, bgv,
              cloc, cidx, tmp,
              fbr, fbg, fbb, fba, fbd,
              sem_pix0, sem_pay0, sem_pix1, sem_pay1, sem_out):
    wid = lax.axis_index("s") * 2 + lax.axis_index("c")
    lo = wid * _PPW

    bufs = [
        (pixv0, crv0, cgv0, cbv0, av0, dv0, sem_pix0, sem_pay0),
        (pixv1, crv1, cgv1, cbv1, av1, dv1, sem_pix1, sem_pay1),
    ]
    hbm_in = (cr_hbm, cg_hbm, cb_hbm, a_hbm, d_hbm)

    def fire(sb, parity):
        off = sb * _SB
        pv, cr, cg, cb, av_, dv_, s_pix, s_pay = bufs[parity]
        hp = pltpu.async_copy(pix_hbm.at[pl.ds(off, _SB)], pv, s_pix)
        hs = [pltpu.async_copy(src.at[pl.ds(off, _SB)], dst, s_pay)
              for src, dst in zip(hbm_in, (cr, cg, cb, av_, dv_))]
        return hp, hs

    handles = fire(0, 0)

    pltpu.sync_copy(bg_hbm, bgv)
    bgvec = bgv[...]
    bg_r = bgvec[0]
    bg_g = bgvec[1]
    bg_b = bgvec[2]

    def init_body(j, c):
        o = j * 16
        fbr[pl.ds(o, 16)] = jnp.full((16,), bg_r, jnp.float32)
        fbg[pl.ds(o, 16)] = jnp.full((16,), bg_g, jnp.float32)
        fbb[pl.ds(o, 16)] = jnp.full((16,), bg_b, jnp.float32)
        fba[pl.ds(o, 16)] = jnp.zeros((16,), jnp.float32)
        fbd[pl.ds(o, 16)] = jnp.zeros((16,), jnp.float32)
        return c

    lax.fori_loop(0, _PPW // 16, init_body, 0)

    lanes = lax.iota(jnp.int32, 16)

    for sb in range(_N // _SB):
        parity = sb % 2
        pixv, crv, cgv, cbv, av, dv = bufs[parity][:6]
        hp, hs = handles
        if sb + 1 < _N // _SB:
            handles = fire(sb + 1, 1 - parity)

        hp.wait()

        def scan_body(k, cnt):
            base = k * 16
            pv = pixv[pl.ds(base, 16)]
            d0 = pv - lo
            owned = plsc.bitcast(d0, jnp.uint32) < jnp.uint32(_PPW)
            plsc.store_compressed(cloc.at[pl.ds(cnt, 16)], d0, mask=owned)
            plsc.store_compressed(cidx.at[pl.ds(cnt, 16)], base + lanes,
                                  mask=owned)
            return cnt + plsc.all_reduce_population_count(owned)[0]

        cnt = lax.fori_loop(0, _SB_CHUNKS, scan_body, 0, unroll=4)

        for h in hs:
            h.wait()

        def blend_body(t, c):
            base = t * 16
            m = (base + lanes) < cnt
            local = jnp.clip(cloc[pl.ds(base, 16)], 0, _PPW - 1)
            gi = jnp.clip(cidx[pl.ds(base, 16)], 0, _SB - 1)
            r = plsc.load_gather(crv, [gi], mask=m)
            g = plsc.load_gather(cgv, [gi], mask=m)
            b = plsc.load_gather(cbv, [gi], mask=m)
            a = plsc.load_gather(av, [gi], mask=m)
            d = plsc.load_gather(dv, [gi], mask=m)
            one_m_a = 1.0 - a

            def blend_masked(mj):
                cur = plsc.load_gather(fbr, [local], mask=mj)
                plsc.store_scatter(fbr, [local], a * r + one_m_a * cur,
                                   mask=mj)
                cur = plsc.load_gather(fbg, [local], mask=mj)
                plsc.store_scatter(fbg, [local], a * g + one_m_a * cur,
                                   mask=mj)
                cur = plsc.load_gather(fbb, [local], mask=mj)
                plsc.store_scatter(fbb, [local], a * b + one_m_a * cur,
                                   mask=mj)
                cur = plsc.load_gather(fba, [local], mask=mj)
                plsc.store_scatter(fba, [local], a + one_m_a * cur,
                                   mask=mj)
                plsc.store_scatter(fbd, [local], d, mask=mj)

            plsc.store_scatter(tmp, [local], lanes, mask=m)
            back = plsc.load_gather(tmp, [local], mask=m)
            ndup = plsc.all_reduce_population_count((back != lanes) & m)[0]

            @pl.when(ndup == 0)
            def _():
                blend_masked(m)

            @pl.when(ndup > 0)
            def _():
                def lane_body(j, cc):
                    mj = m & (lanes == j)
                    n_j = plsc.all_reduce_population_count(mj)[0]

                    @pl.when(n_j > 0)
                    def _():
                        blend_masked(mj)

                    return cc

                lax.fori_loop(0, 16, lane_body, 0)

            return c

        lax.fori_loop(0, (cnt + 15) // 16, blend_body, 0)

    hw = [
        pltpu.async_copy(fbr, outc.at[pl.ds(lo, _PPW)], sem_out),
        pltpu.async_copy(fbg, outc.at[pl.ds(_NPIX + lo, _PPW)], sem_out),
        pltpu.async_copy(fbb, outc.at[pl.ds(2 * _NPIX + lo, _PPW)], sem_out),
        pltpu.async_copy(fba, outa.at[pl.ds(lo, _PPW)], sem_out),
        pltpu.async_copy(fbd, outd.at[pl.ds(lo, _PPW)], sem_out),
    ]
    for h in hw:
        h.wait()


def kernel(xyz, features, opacity, image_height, image_width, bg_color):
    wh = jnp.stack([image_width, image_height]).astype(jnp.float32)
    xs = xyz[:, 0].reshape(128, 128)
    ys = xyz[:, 1].reshape(128, 128)
    r0 = features[:, 0, 0].reshape(128, 128)
    g0 = features[:, 0, 1].reshape(128, 128)
    b0 = features[:, 0, 2].reshape(128, 128)

    pix, cr, cg, cb = pl.pallas_call(
        _prep_kernel,
        in_specs=[pl.BlockSpec(memory_space=pltpu.SMEM)]
        + [pl.BlockSpec((128, 128), lambda: (0, 0))] * 5,
        out_shape=[
            jax.ShapeDtypeStruct((128, 128), jnp.int32),
            jax.ShapeDtypeStruct((128, 128), jnp.float32),
            jax.ShapeDtypeStruct((128, 128), jnp.float32),
            jax.ShapeDtypeStruct((128, 128), jnp.float32),
        ],
    )(wh, xs, ys, r0, g0, b0)

    bg16 = jnp.concatenate([bg_color, jnp.zeros((13,), jnp.float32)])

    blend = functools.partial(
        pl.kernel,
        out_type=[
            jax.ShapeDtypeStruct((3 * _NPIX,), jnp.float32),
            jax.ShapeDtypeStruct((_NPIX,), jnp.float32),
            jax.ShapeDtypeStruct((_NPIX,), jnp.float32),
        ],
        mesh=plsc.VectorSubcoreMesh(core_axis_name="c", subcore_axis_name="s",
                                    num_cores=2, num_subcores=16),
        compiler_params=pltpu.CompilerParams(needs_layout_passes=False),
        scratch_types=(
            [pltpu.VMEM((_SB,), jnp.int32)]
            + [pltpu.VMEM((_SB,), jnp.float32)] * 5
            + [pltpu.VMEM((_SB,), jnp.int32)]
            + [pltpu.VMEM((_SB,), jnp.float32)] * 5
            + [pltpu.VMEM((16,), jnp.float32)]
            + [pltpu.VMEM((_CB,), jnp.int32)] * 2
            + [pltpu.VMEM((_PPW,), jnp.int32)]
            + [pltpu.VMEM((_PPW,), jnp.float32)] * 5
            + [pltpu.SemaphoreType.DMA] * 5
        ),
    )(_sc_blend)

    outc, outa, outd = blend(
        pix.reshape(_N), cr.reshape(_N), cg.reshape(_N), cb.reshape(_N),
        opacity[:, 0], xyz[:, 2], bg16)

    color_img = outc.reshape(3, _HW, _HW)
    depth_img = outd.reshape(1, _HW, _HW)
    alpha_img = outa.reshape(1, _HW, _HW)
    return color_img, depth_img, alpha_img


# speedup vs baseline: 3601.8405x; 1.0158x over previous
import functools

import jax
import jax.numpy as jnp
from jax import lax
from jax.experimental import pallas as pl
from jax.experimental.pallas import tpu as pltpu
from jax.experimental.pallas import tpu_sc as plsc

_N = 16384
_HW = 384
_NPIX = _HW * _HW
_NW = 32
_PPW = _NPIX // _NW
_SENT = 1 << 20
_CHUNKS = _N // 16


def _prep_kernel(wh_ref, xs_ref, ys_ref, r_ref, g_ref, b_ref,
                 pix_ref, cr_ref, cg_ref, cb_ref):
    x = xs_ref[...]
    y = ys_ref[...]
    wf = wh_ref[0]
    hf = wh_ref[1]
    wi = wf.astype(jnp.int32)
    hi = hf.astype(jnp.int32)

    xmin = jnp.min(x)
    xmax = jnp.max(x)
    ymin = jnp.min(y)
    ymax = jnp.max(y)

    xn = (x - xmin) / (xmax - xmin + 1e-08)
    yn = (y - ymin) / (ymax - ymin + 1e-08)
    xi = (xn * wf).astype(jnp.int32)
    yi = (yn * hf).astype(jnp.int32)
    valid = (xi >= 0) & (xi < wi) & (yi >= 0) & (yi < hi)
    xc = jnp.clip(xi, 0, wi - 1)
    yc = jnp.clip(yi, 0, hi - 1)

    p = yc * _HW + xc
    pix_ref[...] = jnp.where(valid, p, _SENT)
    cr_ref[...] = jnp.clip(r_ref[...] + 0.5, 0.0, 1.0)
    cg_ref[...] = jnp.clip(g_ref[...] + 0.5, 0.0, 1.0)
    cb_ref[...] = jnp.clip(b_ref[...] + 0.5, 0.0, 1.0)


_SB = 4096
_SB_CHUNKS = _SB // 16
_CB = _SB + 16


def _sc_blend(pix_hbm, cr_hbm, cg_hbm, cb_hbm, a_hbm, d_hbm, bg_hbm,
              outc, outa, outd,
              pixv0, crv0, cgv0, cbv0, av0, dv0,
              pixv1, crv1, cgv1, cbv1, av1, dv1, bgv,
              cloc, cidx, tmp,
              fbr, fbg, fbb, fba, fbd,
              sem_pix0, sem_pay0, sem_pix1, sem_pay1, sem_out):
    wid = lax.axis_index("s") * 2 + lax.axis_index("c")
    lo = wid * _PPW

    bufs = [
        (pixv0, crv0, cgv0, cbv0, av0, dv0, sem_pix0, sem_pay0),
        (pixv1, crv1, cgv1, cbv1, av1, dv1, sem_pix1, sem_pay1),
    ]
    hbm_in = (cr_hbm, cg_hbm, cb_hbm, a_hbm, d_hbm)

    def fire(sb, parity):
        off = sb * _SB
        pv, cr, cg, cb, av_, dv_, s_pix, s_pay = bufs[parity]
        hp = pltpu.async_copy(pix_hbm.at[pl.ds(off, _SB)], pv, s_pix)
        hs = [pltpu.async_copy(src.at[pl.ds(off, _SB)], dst, s_pay)
              for src, dst in zip(hbm_in, (cr, cg, cb, av_, dv_))]
        return hp, hs

    handles = fire(0, 0)

    pltpu.sync_copy(bg_hbm, bgv)
    bgvec = bgv[...]
    bg_r = bgvec[0]
    bg_g = bgvec[1]
    bg_b = bgvec[2]

    def init_body(j, c):
        o = j * 16
        fbr[pl.ds(o, 16)] = jnp.full((16,), bg_r, jnp.float32)
        fbg[pl.ds(o, 16)] = jnp.full((16,), bg_g, jnp.float32)
        fbb[pl.ds(o, 16)] = jnp.full((16,), bg_b, jnp.float32)
        fba[pl.ds(o, 16)] = jnp.zeros((16,), jnp.float32)
        fbd[pl.ds(o, 16)] = jnp.zeros((16,), jnp.float32)
        return c

    lax.fori_loop(0, _PPW // 16, init_body, 0, unroll=4)

    lanes = lax.iota(jnp.int32, 16)

    for sb in range(_N // _SB):
        parity = sb % 2
        pixv, crv, cgv, cbv, av, dv = bufs[parity][:6]
        hp, hs = handles
        if sb + 1 < _N // _SB:
            handles = fire(sb + 1, 1 - parity)

        hp.wait()

        def scan_body(k, cnt):
            base = k * 16
            pv = pixv[pl.ds(base, 16)]
            d0 = pv - lo
            owned = plsc.bitcast(d0, jnp.uint32) < jnp.uint32(_PPW)
            plsc.store_compressed(cloc.at[pl.ds(cnt, 16)], d0, mask=owned)
            plsc.store_compressed(cidx.at[pl.ds(cnt, 16)], base + lanes,
                                  mask=owned)
            return cnt + plsc.all_reduce_population_count(owned)[0]

        cnt = lax.fori_loop(0, _SB_CHUNKS, scan_body, 0, unroll=8)

        for h in hs:
            h.wait()

        def blend_body(t, c):
            base = t * 16
            m = (base + lanes) < cnt
            local = jnp.clip(cloc[pl.ds(base, 16)], 0, _PPW - 1)
            gi = jnp.clip(cidx[pl.ds(base, 16)], 0, _SB - 1)
            r = plsc.load_gather(crv, [gi], mask=m)
            g = plsc.load_gather(cgv, [gi], mask=m)
            b = plsc.load_gather(cbv, [gi], mask=m)
            a = plsc.load_gather(av, [gi], mask=m)
            d = plsc.load_gather(dv, [gi], mask=m)
            one_m_a = 1.0 - a

            def blend_masked(mj):
                cur = plsc.load_gather(fbr, [local], mask=mj)
                plsc.store_scatter(fbr, [local], a * r + one_m_a * cur,
                                   mask=mj)
                cur = plsc.load_gather(fbg, [local], mask=mj)
                plsc.store_scatter(fbg, [local], a * g + one_m_a * cur,
                                   mask=mj)
                cur = plsc.load_gather(fbb, [local], mask=mj)
                plsc.store_scatter(fbb, [local], a * b + one_m_a * cur,
                                   mask=mj)
                cur = plsc.load_gather(fba, [local], mask=mj)
                plsc.store_scatter(fba, [local], a + one_m_a * cur,
                                   mask=mj)
                plsc.store_scatter(fbd, [local], d, mask=mj)

            plsc.store_scatter(tmp, [local], lanes, mask=m)
            back = plsc.load_gather(tmp, [local], mask=m)
            ndup = plsc.all_reduce_population_count((back != lanes) & m)[0]

            @pl.when(ndup == 0)
            def _():
                blend_masked(m)

            @pl.when(ndup > 0)
            def _():
                def lane_body(j, cc):
                    mj = m & (lanes == j)
                    n_j = plsc.all_reduce_population_count(mj)[0]

                    @pl.when(n_j > 0)
                    def _():
                        blend_masked(mj)

                    return cc

                lax.fori_loop(0, 16, lane_body, 0)

            return c

        lax.fori_loop(0, (cnt + 15) // 16, blend_body, 0)

    hw = [
        pltpu.async_copy(fbr, outc.at[pl.ds(lo, _PPW)], sem_out),
        pltpu.async_copy(fbg, outc.at[pl.ds(_NPIX + lo, _PPW)], sem_out),
        pltpu.async_copy(fbb, outc.at[pl.ds(2 * _NPIX + lo, _PPW)], sem_out),
        pltpu.async_copy(fba, outa.at[pl.ds(lo, _PPW)], sem_out),
        pltpu.async_copy(fbd, outd.at[pl.ds(lo, _PPW)], sem_out),
    ]
    for h in hw:
        h.wait()


def kernel(xyz, features, opacity, image_height, image_width, bg_color):
    wh = jnp.stack([image_width, image_height]).astype(jnp.float32)
    xs = xyz[:, 0].reshape(128, 128)
    ys = xyz[:, 1].reshape(128, 128)
    r0 = features[:, 0, 0].reshape(128, 128)
    g0 = features[:, 0, 1].reshape(128, 128)
    b0 = features[:, 0, 2].reshape(128, 128)

    pix, cr, cg, cb = pl.pallas_call(
        _prep_kernel,
        in_specs=[pl.BlockSpec(memory_space=pltpu.SMEM)]
        + [pl.BlockSpec((128, 128), lambda: (0, 0))] * 5,
        out_shape=[
            jax.ShapeDtypeStruct((128, 128), jnp.int32),
            jax.ShapeDtypeStruct((128, 128), jnp.float32),
            jax.ShapeDtypeStruct((128, 128), jnp.float32),
            jax.ShapeDtypeStruct((128, 128), jnp.float32),
        ],
    )(wh, xs, ys, r0, g0, b0)

    bg16 = jnp.concatenate([bg_color, jnp.zeros((13,), jnp.float32)])

    blend = functools.partial(
        pl.kernel,
        out_type=[
            jax.ShapeDtypeStruct((3 * _NPIX,), jnp.float32),
            jax.ShapeDtypeStruct((_NPIX,), jnp.float32),
            jax.ShapeDtypeStruct((_NPIX,), jnp.float32),
        ],
        mesh=plsc.VectorSubcoreMesh(core_axis_name="c", subcore_axis_name="s",
                                    num_cores=2, num_subcores=16),
        compiler_params=pltpu.CompilerParams(needs_layout_passes=False),
        scratch_types=(
            [pltpu.VMEM((_SB,), jnp.int32)]
            + [pltpu.VMEM((_SB,), jnp.float32)] * 5
            + [pltpu.VMEM((_SB,), jnp.int32)]
            + [pltpu.VMEM((_SB,), jnp.float32)] * 5
            + [pltpu.VMEM((16,), jnp.float32)]
            + [pltpu.VMEM((_CB,), jnp.int32)] * 2
            + [pltpu.VMEM((_PPW,), jnp.int32)]
            + [pltpu.VMEM((_PPW,), jnp.float32)] * 5
            + [pltpu.SemaphoreType.DMA] * 5
        ),
    )(_sc_blend)

    outc, outa, outd = blend(
        pix.reshape(_N), cr.reshape(_N), cg.reshape(_N), cb.reshape(_N),
        opacity[:, 0], xyz[:, 2], bg16)

    color_img = outc.reshape(3, _HW, _HW)
    depth_img = outd.reshape(1, _HW, _HW)
    alpha_img = outa.reshape(1, _HW, _HW)
    return color_img, depth_img, alpha_img
